# baseline (speedup 1.0000x reference)
import functools

import numpy as np

import jax
import jax.numpy as jnp
from jax import lax
from jax.experimental import pallas as pl
from jax.experimental.pallas import tpu as pltpu
from jax.experimental.pallas import tpu_sc as plsc

_NC = 2
_NS = 16
_NW = _NC * _NS
_L = 16

_B = 80
_ROW_BLK = 1000


def _tc_build_tables(h, pos, p, pos_p, Wq_s, bq_s, Wk, bk, Wv, bv, NH):
    N, C = h.shape
    M = p.shape[0]
    HD = Wq_s.shape[1]
    QW = NH * _L // 2 + 16
    KVW = NH * _L + 16

    def pack_pair(x, o):
        ua = lax.bitcast_convert_type(
            x[:, o:o + _L].astype(jnp.bfloat16), jnp.uint16).astype(jnp.uint32)
        ub = lax.bitcast_convert_type(
            x[:, o + _L:o + 2 * _L].astype(jnp.bfloat16),
            jnp.uint16).astype(jnp.uint32)
        return lax.bitcast_convert_type(ua | (ub << 16), jnp.float32)

    def body(h_ref, pos_ref, p_ref, posp_ref, wq_ref, bq_ref, wk_ref, bk_ref,
             wv_ref, bv_ref, qa_ref, qb_ref, kva_ref, kvb_ref):
        hb = h_ref[...]
        q = jnp.dot(hb, wq_ref[...], preferred_element_type=jnp.float32) + bq_ref[...]
        po = pos_ref[...]
        n2 = jnp.sum(po * po, axis=1, keepdims=True)
        one = jnp.ones_like(n2)
        zpad = jnp.zeros((hb.shape[0], 11), jnp.float32)
        qext = jnp.concatenate([po, n2, one, zpad], axis=1)
        NHW = NH * _L
        qa_ref[...] = jnp.concatenate(
            [pack_pair(q, 2 * _L * pr) for pr in range(NH // 2)] + [qext],
            axis=1)
        qb_ref[...] = jnp.concatenate(
            [pack_pair(q, NHW + 2 * _L * pr) for pr in range(NH // 2)] + [qext],
            axis=1)

        pb = p_ref[...]
        k = jnp.dot(pb, wk_ref[...], preferred_element_type=jnp.float32) + bk_ref[...]
        v = jnp.dot(pb, wv_ref[...], preferred_element_type=jnp.float32) + bv_ref[...]
        pp = posp_ref[...]
        pn2 = jnp.sum(pp * pp, axis=1, keepdims=True)
        pone = jnp.ones_like(pn2)
        kext = jnp.concatenate([-2.0 * pp, pone, pn2, zpad], axis=1)
        kva_ref[...] = jnp.concatenate(
            [pack_pair(k, 2 * _L * pr) for pr in range(NH // 2)] + [kext]
            + [pack_pair(v, 2 * _L * pr) for pr in range(NH // 2)], axis=1)
        kvb_ref[...] = jnp.concatenate(
            [pack_pair(k, NHW + 2 * _L * pr) for pr in range(NH // 2)] + [kext]
            + [pack_pair(v, NHW + 2 * _L * pr) for pr in range(NH // 2)],
            axis=1)

    grid = (N // _ROW_BLK,)
    full = lambda s: pl.BlockSpec(s, lambda i: (0,) * len(s))
    rows = lambda w: pl.BlockSpec((_ROW_BLK, w), lambda i: (i, 0))
    return pl.pallas_call(
        body,
        grid=grid,
        in_specs=[rows(C), rows(3), rows(C), rows(3),
                  full((C, HD)), full((1, HD)), full((C, HD)), full((1, HD)),
                  full((C, HD)), full((1, HD))],
        out_specs=[rows(QW), rows(QW), rows(KVW), rows(KVW)],
        out_shape=[jax.ShapeDtypeStruct((N, QW), jnp.float32),
                   jax.ShapeDtypeStruct((N, QW), jnp.float32),
                   jax.ShapeDtypeStruct((M, KVW), jnp.float32),
                   jax.ShapeDtypeStruct((M, KVW), jnp.float32)],
    )(h, pos, p, pos_p, Wq_s, bq_s.reshape(1, -1), Wk, bk.reshape(1, -1),
      Wv, bv.reshape(1, -1))


def _sc_edge_pass(qt, kvt, row, col, WcTp, step, coeff, n_pad, NH):
    NB = row.shape[1]
    EDIM = WcTp.shape[1]
    NEC = EDIM // _L
    NP = NH // 2
    RW = NH * _L + 16
    QW = NP * _L + 16
    KVW = 2 * NP * _L + 16
    RPT = n_pad // _NS
    NZ = 8 if RPT % 8 == 0 else 5
    ZC = RPT // NZ

    mesh = plsc.VectorSubcoreMesh(core_axis_name="c", subcore_axis_name="s",
                                  num_cores=_NC, num_subcores=_NS)

    @functools.partial(
        pl.kernel,
        out_type=jax.ShapeDtypeStruct((_NC, n_pad, RW), jnp.float32),
        mesh=mesh,
        compiler_params=pltpu.CompilerParams(use_tc_tiling_on_sc=False,
                                             needs_layout_passes=False),
        scratch_types=[
            pltpu.VMEM((NB, _B), jnp.int32),
            pltpu.VMEM((NB, _B), jnp.int32),
            pltpu.VMEM((2, _B, QW), jnp.float32),
            pltpu.VMEM((2, _B, KVW), jnp.float32),
            pltpu.VMEM((2, _B, RW), jnp.float32),
            pltpu.VMEM((ZC, RW), jnp.float32),
            pltpu.VMEM((NH, EDIM), jnp.float32),
            pltpu.VMEM_SHARED((n_pad, RW), jnp.float32),
            pltpu.SemaphoreType.DMA,
            pltpu.SemaphoreType.DMA,
            pltpu.SemaphoreType.DMA,
            pltpu.SemaphoreType.DMA,
        ],
    )
    def edge_kernel(qt_hbm, kvt_hbm, row_hbm, col_hbm, wct_hbm, out_hbm,
                    rix_all, cix_all, qrows2, kvrows2, contrib2, stage, wct_v,
                    acc_sh, sg0, sg1, ss0, ss1):
        cid = lax.axis_index("c")
        sid = lax.axis_index("s")
        wid = sid * _NC + cid
        lane_i = lax.iota(jnp.int32, _L)
        lane = lane_i.astype(jnp.float32)
        sg = [sg0, sg1]
        ss = [ss0, ss1]

        pltpu.sync_copy(wct_hbm, wct_v)
        wct = [[wct_v[hh, pl.ds(j * _L, _L)] for j in range(NEC)]
               for hh in range(NH)]
        offs = [(lane + float(j * _L)) * step for j in range(NEC)]

        pltpu.sync_copy(row_hbm.at[wid], rix_all)
        pltpu.sync_copy(col_hbm.at[wid], cix_all)

        def zbody(i, _):
            r = i // (RW // _L)
            c = i % (RW // _L)
            stage[r, pl.ds(c * _L, _L)] = jnp.zeros((_L,), jnp.float32)
            return 0
        lax.fori_loop(0, ZC * (RW // _L), zbody, 0)
        tb = sid * RPT
        for k in range(NZ):
            pltpu.sync_copy(stage, acc_sh.at[pl.ds(tb + k * ZC, ZC)])
        plsc.subcore_barrier()

        def gathers(bi, pp):
            return (pltpu.make_async_copy(qt_hbm.at[rix_all.at[bi]],
                                          qrows2.at[pp], sg[pp]),
                    pltpu.make_async_copy(kvt_hbm.at[cix_all.at[bi]],
                                          kvrows2.at[pp], sg[pp]))

        def scatter(bi, pp):
            return pltpu.make_async_copy(contrib2.at[pp],
                                         acc_sh.at[rix_all.at[bi]], ss[pp])

        def issue_gathers(bi, pp):
            for c in gathers(bi, pp):
                c.start()

        def compute(pp):
            qr = qrows2.at[pp]
            kvr = kvrows2.at[pp]
            ctr = contrib2.at[pp]

            def unpk(w):
                return plsc.unpack(plsc.bitcast(w, jnp.bfloat16),
                                   format=plsc.PackFormat.INTERLEAVED)

            @plsc.parallel_loop(0, _B, 1, unroll=8)
            def edge(i):
                qe = qr[i, pl.ds(NP * _L, _L)]
                ke = kvr[i, pl.ds(NP * _L, _L)]
                dist = jnp.sum(qe * ke, axis=0)
                rbf = []
                for j in range(NEC):
                    d = dist - offs[j]
                    rbf.append(jnp.exp(coeff * (d * d)))
                qk = []
                for pr in range(NP):
                    qa_, qb_ = unpk(qr[i, pl.ds(pr * _L, _L)])
                    ka_, kb_ = unpk(kvr[i, pl.ds(pr * _L, _L)])
                    qk.append(qa_ * ka_)
                    qk.append(qb_ * kb_)
                den = jnp.zeros((_L,), jnp.float32)
                wbs = []
                for hh in range(NH):
                    t = qk[hh] + rbf[0] * wct[hh][0]
                    for j in range(1, NEC):
                        t = t + rbf[j] * wct[hh][j]
                    s = jnp.sum(t, axis=0)
                    wb = jnp.exp(jnp.full((_L,), s))
                    wbs.append(wb)
                    den = jnp.where(lane_i == hh, wb, den)
                for pr in range(NP):
                    va_, vb_ = unpk(kvr[i, pl.ds((NP + 1 + pr) * _L, _L)])
                    ctr[i, pl.ds(2 * pr * _L, _L)] = va_ * wbs[2 * pr]
                    ctr[i, pl.ds((2 * pr + 1) * _L, _L)] = vb_ * wbs[2 * pr + 1]
                ctr[i, pl.ds(NH * _L, _L)] = den

        issue_gathers(0, 0)

        def group(g, _):
            for pp in range(2):
                bi = 2 * g + pp

                @pl.when(bi < NB)
                def _():
                    @pl.when(bi + 1 < NB)
                    def _():
                        issue_gathers(bi + 1, 1 - pp)
                    for c in gathers(bi, pp):
                        c.wait()

                    @pl.when(bi >= 2)
                    def _():
                        scatter(bi - 2, pp).wait()
                    compute(pp)
                    scatter(bi, pp).start(add=True)
            return 0
        lax.fori_loop(0, (NB + 1) // 2, group, 0)
        scatter(NB - 2, (NB - 2) % 2).wait()
        scatter(NB - 1, (NB - 1) % 2).wait()

        plsc.subcore_barrier()
        for k in range(NZ):
            pltpu.sync_copy(acc_sh.at[pl.ds(tb + k * ZC, ZC)], stage)
            pltpu.sync_copy(stage, out_hbm.at[cid, pl.ds(tb + k * ZC, ZC)])

    return edge_kernel(qt, kvt, row, col, WcTp)


def _tc_output(acc_a, acc_b, Wo, bo, W1, b1, W2, b2, Erep, n_rows, NH):
    C = Wo.shape[1]
    HD = Wo.shape[0]
    C2 = W1.shape[1]
    RW = acc_a.shape[2]
    NCQ = acc_a.shape[0]
    NHW = NH * _L

    def body(acca_ref, accb_ref, wo_ref, bo_ref, w1_ref, b1_ref, w2_ref,
             b2_ref, erep_ref, out_ref):
        sa = jnp.sum(acca_ref[...], axis=0)
        sb = jnp.sum(accb_ref[...], axis=0)
        num = jnp.concatenate([sa[:, :NHW], sb[:, :NHW]], axis=1)
        den = jnp.concatenate([sa[:, NHW:NHW + NH], sb[:, NHW:NHW + NH]],
                              axis=1) + 1e-16
        den_wide = jnp.dot(den, erep_ref[...],
                           preferred_element_type=jnp.float32)
        agg = num / den_wide
        x = jnp.dot(agg, wo_ref[...], preferred_element_type=jnp.float32) + bo_ref[...]
        hid = jnp.dot(x, w1_ref[...], preferred_element_type=jnp.float32) + b1_ref[...]
        hid = hid * (1.0 / (1.0 + jnp.exp(-hid)))
        out_ref[...] = jnp.dot(hid, w2_ref[...],
                               preferred_element_type=jnp.float32) + b2_ref[...]

    grid = (n_rows // _ROW_BLK,)
    full = lambda s: pl.BlockSpec(s, lambda i: (0,) * len(s))
    accs = pl.BlockSpec((NCQ, _ROW_BLK, RW), lambda i: (0, i, 0))
    return pl.pallas_call(
        body,
        grid=grid,
        in_specs=[accs, accs,
                  full((HD, C)), full((1, C)), full((C, C2)), full((1, C2)),
                  full((C2, C)), full((1, C)), full((2 * NH, HD))],
        out_specs=pl.BlockSpec((_ROW_BLK, C), lambda i: (i, 0)),
        out_shape=jax.ShapeDtypeStruct((n_rows, C), jnp.float32),
    )(acc_a, acc_b, Wo, bo.reshape(1, -1), W1, b1.reshape(1, -1), W2,
      b2.reshape(1, -1), Erep)


def kernel(pos, h, edge_index, pos_p, p, W_edge, b_edge, Wq, bq, Wk, bk,
           Wv, bv, We, be, Wo, bo, W1, b1, W2, b2):
    N, C = h.shape
    M = p.shape[0]
    E = edge_index.shape[1]
    EDIM = W_edge.shape[0]
    H = We.shape[1]
    DH = Wq.shape[1] // H
    NH = H // 2

    scale = 1.0 / float(np.sqrt(DH))
    Wq_s = Wq * scale
    bq_s = bq * scale
    WcT = (W_edge @ We).T
    offs_np = np.linspace(0.0, 30.0, EDIM)
    step = float(offs_np[1] - offs_np[0])
    coeff = float(-0.5 / step ** 2)

    qa, qb, kva, kvb = _tc_build_tables(h, pos, p, pos_p, Wq_s, bq_s, Wk, bk,
                                        Wv, bv, NH)

    tile = _NW * _B
    E_pad = ((E + tile - 1) // tile) * tile
    row = edge_index[0]
    col = edge_index[1]
    if E_pad != E:
        row = jnp.concatenate([row, jnp.full((E_pad - E,), N, jnp.int32)])
        col = jnp.concatenate([col, jnp.zeros((E_pad - E,), jnp.int32)])
    row = row.reshape(_NW, E_pad // (_NW * _B), _B)
    col = col.reshape(_NW, E_pad // (_NW * _B), _B)
    if E_pad == E and N % _NS == 0:
        n_pad = N
    else:
        n_pad = ((N + 1 + 16 * _NS - 1) // (16 * _NS)) * (16 * _NS)

    acc_a = _sc_edge_pass(qa, kva, row, col, WcT[:NH], step, coeff, n_pad, NH)
    acc_b = _sc_edge_pass(qb, kvb, row, col, WcT[NH:], step, coeff, n_pad, NH)

    lane_h = jnp.arange(H * DH) // DH
    Erep = (lane_h[None, :] == jnp.arange(H)[:, None]).astype(jnp.float32)
    if n_pad != N:
        acc_a = acc_a[:, :N]
        acc_b = acc_b[:, :N]
    return _tc_output(acc_a, acc_b, Wo, bo, W1, b1, W2, b2, Erep, N, NH)


# speedup vs baseline: 1.4909x; 1.4909x over previous
import functools

import numpy as np

import jax
import jax.numpy as jnp
from jax import lax
from jax.experimental import pallas as pl
from jax.experimental.pallas import tpu as pltpu
from jax.experimental.pallas import tpu_sc as plsc

_NC = 2
_NS = 16
_NW = _NC * _NS
_L = 16

_B = 80
_ROW_BLK = 1000


def _tc_build_tables(h, pos, p, pos_p, Wq_s, bq_s, Wk, bk, Wv, bv, NH):
    N, C = h.shape
    M = p.shape[0]
    HD = Wq_s.shape[1]
    QW = NH * _L // 2 + 16
    KVW = NH * _L + 16

    def pack_pair(x, o):
        ua = lax.bitcast_convert_type(
            x[:, o:o + _L].astype(jnp.bfloat16), jnp.uint16).astype(jnp.uint32)
        ub = lax.bitcast_convert_type(
            x[:, o + _L:o + 2 * _L].astype(jnp.bfloat16),
            jnp.uint16).astype(jnp.uint32)
        return lax.bitcast_convert_type(ua | (ub << 16), jnp.float32)

    def body(h_ref, pos_ref, p_ref, posp_ref, wq_ref, bq_ref, wk_ref, bk_ref,
             wv_ref, bv_ref, qa_ref, qb_ref, kva_ref, kvb_ref):
        hb = h_ref[...]
        q = jnp.dot(hb, wq_ref[...], preferred_element_type=jnp.float32) + bq_ref[...]
        po = pos_ref[...]
        n2 = jnp.sum(po * po, axis=1, keepdims=True)
        one = jnp.ones_like(n2)
        zpad = jnp.zeros((hb.shape[0], 11), jnp.float32)
        qext = jnp.concatenate([po, n2, one, zpad], axis=1)
        NHW = NH * _L
        qa_ref[...] = jnp.concatenate(
            [pack_pair(q, 2 * _L * pr) for pr in range(NH // 2)] + [qext],
            axis=1)
        qb_ref[...] = jnp.concatenate(
            [pack_pair(q, NHW + 2 * _L * pr) for pr in range(NH // 2)] + [qext],
            axis=1)

        pb = p_ref[...]
        k = jnp.dot(pb, wk_ref[...], preferred_element_type=jnp.float32) + bk_ref[...]
        v = jnp.dot(pb, wv_ref[...], preferred_element_type=jnp.float32) + bv_ref[...]
        pp = posp_ref[...]
        pn2 = jnp.sum(pp * pp, axis=1, keepdims=True)
        pone = jnp.ones_like(pn2)
        kext = jnp.concatenate([-2.0 * pp, pone, pn2, zpad], axis=1)
        kva_ref[...] = jnp.concatenate(
            [pack_pair(k, 2 * _L * pr) for pr in range(NH // 2)] + [kext]
            + [pack_pair(v, 2 * _L * pr) for pr in range(NH // 2)], axis=1)
        kvb_ref[...] = jnp.concatenate(
            [pack_pair(k, NHW + 2 * _L * pr) for pr in range(NH // 2)] + [kext]
            + [pack_pair(v, NHW + 2 * _L * pr) for pr in range(NH // 2)],
            axis=1)

    grid = (N // _ROW_BLK,)
    full = lambda s: pl.BlockSpec(s, lambda i: (0,) * len(s))
    rows = lambda w: pl.BlockSpec((_ROW_BLK, w), lambda i: (i, 0))
    return pl.pallas_call(
        body,
        grid=grid,
        in_specs=[rows(C), rows(3), rows(C), rows(3),
                  full((C, HD)), full((1, HD)), full((C, HD)), full((1, HD)),
                  full((C, HD)), full((1, HD))],
        out_specs=[rows(QW), rows(QW), rows(KVW), rows(KVW)],
        out_shape=[jax.ShapeDtypeStruct((N, QW), jnp.float32),
                   jax.ShapeDtypeStruct((N, QW), jnp.float32),
                   jax.ShapeDtypeStruct((M, KVW), jnp.float32),
                   jax.ShapeDtypeStruct((M, KVW), jnp.float32)],
    )(h, pos, p, pos_p, Wq_s, bq_s.reshape(1, -1), Wk, bk.reshape(1, -1),
      Wv, bv.reshape(1, -1))


def _sc_edge_pass(qta, kvta, qtb, kvtb, row, col, WcT, step, coeff, n_pad,
                  NH):
    NB = row.shape[1]
    EDIM = WcT.shape[1]
    NEC = EDIM // _L
    NP = NH // 2
    RW = NH * _L + 16
    QW = NP * _L + 16
    KVW = 2 * NP * _L + 16
    RPT = n_pad // _NS
    NZ = 8 if RPT % 8 == 0 else 5
    ZC = RPT // NZ

    mesh = plsc.VectorSubcoreMesh(core_axis_name="c", subcore_axis_name="s",
                                  num_cores=_NC, num_subcores=_NS)

    @functools.partial(
        pl.kernel,
        out_type=[jax.ShapeDtypeStruct((_NC, n_pad, RW), jnp.float32),
                  jax.ShapeDtypeStruct((_NC, n_pad, RW), jnp.float32)],
        mesh=mesh,
        compiler_params=pltpu.CompilerParams(use_tc_tiling_on_sc=False,
                                             needs_layout_passes=False),
        scratch_types=[
            pltpu.VMEM((NB, _B), jnp.int32),
            pltpu.VMEM((NB, _B), jnp.int32),
            pltpu.VMEM((2, _B, QW), jnp.float32),
            pltpu.VMEM((2, _B, KVW), jnp.float32),
            pltpu.VMEM((2, _B, RW), jnp.float32),
            pltpu.VMEM((ZC, RW), jnp.float32),
            pltpu.VMEM((2 * NH, EDIM), jnp.float32),
            pltpu.VMEM_SHARED((n_pad, RW), jnp.float32),
            pltpu.SemaphoreType.DMA,
            pltpu.SemaphoreType.DMA,
            pltpu.SemaphoreType.DMA,
            pltpu.SemaphoreType.DMA,
        ],
    )
    def edge_kernel(qta_hbm, kvta_hbm, qtb_hbm, kvtb_hbm, row_hbm, col_hbm,
                    wct_hbm, outa_hbm, outb_hbm,
                    rix_all, cix_all, qrows2, kvrows2, contrib2, stage, wct_v,
                    acc_sh, sg0, sg1, ss0, ss1):
        cid = lax.axis_index("c")
        sid = lax.axis_index("s")
        wid = sid * _NC + cid
        lane_i = lax.iota(jnp.int32, _L)
        lane = lane_i.astype(jnp.float32)
        sg = [sg0, sg1]
        ss = [ss0, ss1]

        pltpu.sync_copy(wct_hbm, wct_v)
        offs = [(lane + float(j * _L)) * step for j in range(NEC)]

        pltpu.sync_copy(row_hbm.at[wid], rix_all)
        pltpu.sync_copy(col_hbm.at[wid], cix_all)
        tb = sid * RPT

        def zero_acc():
            def zbody(i, _):
                r = i // (RW // _L)
                c = i % (RW // _L)
                stage[r, pl.ds(c * _L, _L)] = jnp.zeros((_L,), jnp.float32)
                return 0
            lax.fori_loop(0, ZC * (RW // _L), zbody, 0)
            for k in range(NZ):
                pltpu.sync_copy(stage, acc_sh.at[pl.ds(tb + k * ZC, ZC)])

        def run_phase(qt_hbm, kvt_hbm, out_hbm, hbase):
            wct = [[wct_v[hbase + hh, pl.ds(j * _L, _L)] for j in range(NEC)]
                   for hh in range(NH)]

            def gathers(bi, pp):
                return (pltpu.make_async_copy(qt_hbm.at[rix_all.at[bi]],
                                              qrows2.at[pp], sg[pp]),
                        pltpu.make_async_copy(kvt_hbm.at[cix_all.at[bi]],
                                              kvrows2.at[pp], sg[pp]))

            def scatter(bi, pp):
                return pltpu.make_async_copy(contrib2.at[pp],
                                             acc_sh.at[rix_all.at[bi]],
                                             ss[pp])

            def issue_gathers(bi, pp):
                for c in gathers(bi, pp):
                    c.start()

            def compute(pp):
                qr = qrows2.at[pp]
                kvr = kvrows2.at[pp]
                ctr = contrib2.at[pp]

                def unpk(w):
                    return plsc.unpack(plsc.bitcast(w, jnp.bfloat16),
                                       format=plsc.PackFormat.INTERLEAVED)

                @plsc.parallel_loop(0, _B, 1, unroll=4)
                def edge(i):
                    qe = qr[i, pl.ds(NP * _L, _L)]
                    ke = kvr[i, pl.ds(NP * _L, _L)]
                    dist = jnp.sum(qe * ke, axis=0)
                    rbf = []
                    for j in range(NEC):
                        d = dist - offs[j]
                        rbf.append(jnp.exp(coeff * (d * d)))
                    qk = []
                    for pr in range(NP):
                        qa_, qb_ = unpk(qr[i, pl.ds(pr * _L, _L)])
                        ka_, kb_ = unpk(kvr[i, pl.ds(pr * _L, _L)])
                        qk.append(qa_ * ka_)
                        qk.append(qb_ * kb_)
                    den = jnp.zeros((_L,), jnp.float32)
                    wbs = []
                    for hh in range(NH):
                        t = qk[hh] + rbf[0] * wct[hh][0]
                        for j in range(1, NEC):
                            t = t + rbf[j] * wct[hh][j]
                        s = jnp.sum(t, axis=0)
                        wb = jnp.exp(jnp.full((_L,), s))
                        wbs.append(wb)
                        den = jnp.where(lane_i == hh, wb, den)
                    for pr in range(NP):
                        va_, vb_ = unpk(kvr[i, pl.ds((NP + 1 + pr) * _L, _L)])
                        ctr[i, pl.ds(2 * pr * _L, _L)] = va_ * wbs[2 * pr]
                        ctr[i, pl.ds((2 * pr + 1) * _L, _L)] = (
                            vb_ * wbs[2 * pr + 1])
                    ctr[i, pl.ds(NH * _L, _L)] = den

            issue_gathers(0, 0)

            def group(g, _):
                for pp in range(2):
                    bi = 2 * g + pp

                    @pl.when(bi < NB)
                    def _():
                        @pl.when(bi + 1 < NB)
                        def _():
                            issue_gathers(bi + 1, 1 - pp)
                        for c in gathers(bi, pp):
                            c.wait()

                        @pl.when(bi >= 2)
                        def _():
                            scatter(bi - 2, pp).wait()
                        compute(pp)
                        scatter(bi, pp).start(add=True)
                return 0
            lax.fori_loop(0, (NB + 1) // 2, group, 0)
            scatter(NB - 2, (NB - 2) % 2).wait()
            scatter(NB - 1, (NB - 1) % 2).wait()

            plsc.subcore_barrier()
            for k in range(NZ):
                pltpu.sync_copy(acc_sh.at[pl.ds(tb + k * ZC, ZC)], stage)
                pltpu.sync_copy(stage, out_hbm.at[cid, pl.ds(tb + k * ZC, ZC)])

        zero_acc()
        plsc.subcore_barrier()
        run_phase(qta_hbm, kvta_hbm, outa_hbm, 0)
        zero_acc()
        plsc.subcore_barrier()
        run_phase(qtb_hbm, kvtb_hbm, outb_hbm, NH)

    return edge_kernel(qta, kvta, qtb, kvtb, row, col, WcT)


def _tc_output(acc_a, acc_b, Wo, bo, W1, b1, W2, b2, Erep, n_rows, NH):
    C = Wo.shape[1]
    HD = Wo.shape[0]
    C2 = W1.shape[1]
    RW = acc_a.shape[2]
    NCQ = acc_a.shape[0]
    NHW = NH * _L

    def body(acca_ref, accb_ref, wo_ref, bo_ref, w1_ref, b1_ref, w2_ref,
             b2_ref, erep_ref, out_ref):
        sa = jnp.sum(acca_ref[...], axis=0)
        sb = jnp.sum(accb_ref[...], axis=0)
        num = jnp.concatenate([sa[:, :NHW], sb[:, :NHW]], axis=1)
        den = jnp.concatenate([sa[:, NHW:NHW + NH], sb[:, NHW:NHW + NH]],
                              axis=1) + 1e-16
        den_wide = jnp.dot(den, erep_ref[...],
                           preferred_element_type=jnp.float32)
        agg = num / den_wide
        x = jnp.dot(agg, wo_ref[...], preferred_element_type=jnp.float32) + bo_ref[...]
        hid = jnp.dot(x, w1_ref[...], preferred_element_type=jnp.float32) + b1_ref[...]
        hid = hid * (1.0 / (1.0 + jnp.exp(-hid)))
        out_ref[...] = jnp.dot(hid, w2_ref[...],
                               preferred_element_type=jnp.float32) + b2_ref[...]

    grid = (n_rows // _ROW_BLK,)
    full = lambda s: pl.BlockSpec(s, lambda i: (0,) * len(s))
    accs = pl.BlockSpec((NCQ, _ROW_BLK, RW), lambda i: (0, i, 0))
    return pl.pallas_call(
        body,
        grid=grid,
        in_specs=[accs, accs,
                  full((HD, C)), full((1, C)), full((C, C2)), full((1, C2)),
                  full((C2, C)), full((1, C)), full((2 * NH, HD))],
        out_specs=pl.BlockSpec((_ROW_BLK, C), lambda i: (i, 0)),
        out_shape=jax.ShapeDtypeStruct((n_rows, C), jnp.float32),
    )(acc_a, acc_b, Wo, bo.reshape(1, -1), W1, b1.reshape(1, -1), W2,
      b2.reshape(1, -1), Erep)


def kernel(pos, h, edge_index, pos_p, p, W_edge, b_edge, Wq, bq, Wk, bk,
           Wv, bv, We, be, Wo, bo, W1, b1, W2, b2):
    N, C = h.shape
    M = p.shape[0]
    E = edge_index.shape[1]
    EDIM = W_edge.shape[0]
    H = We.shape[1]
    DH = Wq.shape[1] // H
    NH = H // 2

    scale = 1.0 / float(np.sqrt(DH))
    Wq_s = Wq * scale
    bq_s = bq * scale
    WcT = (W_edge @ We).T
    offs_np = np.linspace(0.0, 30.0, EDIM)
    step = float(offs_np[1] - offs_np[0])
    coeff = float(-0.5 / step ** 2)

    qa, qb, kva, kvb = _tc_build_tables(h, pos, p, pos_p, Wq_s, bq_s, Wk, bk,
                                        Wv, bv, NH)

    tile = _NW * _B
    E_pad = ((E + tile - 1) // tile) * tile
    row = edge_index[0]
    col = edge_index[1]
    if E_pad != E:
        row = jnp.concatenate([row, jnp.full((E_pad - E,), N, jnp.int32)])
        col = jnp.concatenate([col, jnp.zeros((E_pad - E,), jnp.int32)])
    row = row.reshape(_NW, E_pad // (_NW * _B), _B)
    col = col.reshape(_NW, E_pad // (_NW * _B), _B)
    if E_pad == E and N % _NS == 0:
        n_pad = N
    else:
        n_pad = ((N + 1 + 16 * _NS - 1) // (16 * _NS)) * (16 * _NS)

    acc_a, acc_b = _sc_edge_pass(qa, kva, qb, kvb, row, col, WcT, step,
                                 coeff, n_pad, NH)

    lane_h = jnp.arange(H * DH) // DH
    Erep = (lane_h[None, :] == jnp.arange(H)[:, None]).astype(jnp.float32)
    if n_pad != N:
        acc_a = acc_a[:, :N]
        acc_b = acc_b[:, :N]
    return _tc_output(acc_a, acc_b, Wo, bo, W1, b1, W2, b2, Erep, N, NH)


# speedup vs baseline: 1.5567x; 1.0441x over previous
import functools

import numpy as np

import jax
import jax.numpy as jnp
from jax import lax
from jax.experimental import pallas as pl
from jax.experimental.pallas import tpu as pltpu
from jax.experimental.pallas import tpu_sc as plsc

_NC = 2
_NS = 16
_NW = _NC * _NS
_L = 16

_B = 80
_ROW_BLK = 1000


def _tc_build_tables(h, pos, p, pos_p, Wq_s, bq_s, Wk, bk, Wv, bv, NH):
    N, C = h.shape
    M = p.shape[0]
    HD = Wq_s.shape[1]
    QW = NH * _L // 2 + 16
    KVW = NH * _L + 16

    def pack_pair(x, o):
        ua = lax.bitcast_convert_type(
            x[:, o:o + _L].astype(jnp.bfloat16), jnp.uint16).astype(jnp.uint32)
        ub = lax.bitcast_convert_type(
            x[:, o + _L:o + 2 * _L].astype(jnp.bfloat16),
            jnp.uint16).astype(jnp.uint32)
        return lax.bitcast_convert_type(ua | (ub << 16), jnp.float32)

    def body(h_ref, pos_ref, p_ref, posp_ref, wq_ref, bq_ref, wk_ref, bk_ref,
             wv_ref, bv_ref, qa_ref, qb_ref, kva_ref, kvb_ref):
        hb = h_ref[...]
        q = jnp.dot(hb, wq_ref[...], preferred_element_type=jnp.float32) + bq_ref[...]
        po = pos_ref[...]
        n2 = jnp.sum(po * po, axis=1, keepdims=True)
        one = jnp.ones_like(n2)
        zpad = jnp.zeros((hb.shape[0], 11), jnp.float32)
        qext = jnp.concatenate([po, n2, one, zpad], axis=1)
        NHW = NH * _L
        qa_ref[...] = jnp.concatenate(
            [pack_pair(q, 2 * _L * pr) for pr in range(NH // 2)] + [qext],
            axis=1)
        qb_ref[...] = jnp.concatenate(
            [pack_pair(q, NHW + 2 * _L * pr) for pr in range(NH // 2)] + [qext],
            axis=1)

        pb = p_ref[...]
        k = jnp.dot(pb, wk_ref[...], preferred_element_type=jnp.float32) + bk_ref[...]
        v = jnp.dot(pb, wv_ref[...], preferred_element_type=jnp.float32) + bv_ref[...]
        pp = posp_ref[...]
        pn2 = jnp.sum(pp * pp, axis=1, keepdims=True)
        pone = jnp.ones_like(pn2)
        kext = jnp.concatenate([-2.0 * pp, pone, pn2, zpad], axis=1)
        kva_ref[...] = jnp.concatenate(
            [pack_pair(k, 2 * _L * pr) for pr in range(NH // 2)] + [kext]
            + [pack_pair(v, 2 * _L * pr) for pr in range(NH // 2)], axis=1)
        kvb_ref[...] = jnp.concatenate(
            [pack_pair(k, NHW + 2 * _L * pr) for pr in range(NH // 2)] + [kext]
            + [pack_pair(v, NHW + 2 * _L * pr) for pr in range(NH // 2)],
            axis=1)

    grid = (N // _ROW_BLK,)
    full = lambda s: pl.BlockSpec(s, lambda i: (0,) * len(s))
    rows = lambda w: pl.BlockSpec((_ROW_BLK, w), lambda i: (i, 0))
    return pl.pallas_call(
        body,
        grid=grid,
        in_specs=[rows(C), rows(3), rows(C), rows(3),
                  full((C, HD)), full((1, HD)), full((C, HD)), full((1, HD)),
                  full((C, HD)), full((1, HD))],
        out_specs=[rows(QW), rows(QW), rows(KVW), rows(KVW)],
        out_shape=[jax.ShapeDtypeStruct((N, QW), jnp.float32),
                   jax.ShapeDtypeStruct((N, QW), jnp.float32),
                   jax.ShapeDtypeStruct((M, KVW), jnp.float32),
                   jax.ShapeDtypeStruct((M, KVW), jnp.float32)],
    )(h, pos, p, pos_p, Wq_s, bq_s.reshape(1, -1), Wk, bk.reshape(1, -1),
      Wv, bv.reshape(1, -1))


def _sc_edge_pass(qt, kvt, row, col, WcTp, step, coeff, n_pad, NH):
    NB = row.shape[1]
    EDIM = WcTp.shape[1]
    NEC = EDIM // _L
    NP = NH // 2
    RW = NH * _L + 16
    QW = NP * _L + 16
    KVW = 2 * NP * _L + 16
    RPT = n_pad // _NS
    NZ = 8 if RPT % 8 == 0 else 5
    ZC = RPT // NZ

    mesh = plsc.VectorSubcoreMesh(core_axis_name="c", subcore_axis_name="s",
                                  num_cores=_NC, num_subcores=_NS)

    @functools.partial(
        pl.kernel,
        out_type=jax.ShapeDtypeStruct((_NC, n_pad, RW), jnp.float32),
        mesh=mesh,
        compiler_params=pltpu.CompilerParams(use_tc_tiling_on_sc=False,
                                             needs_layout_passes=False),
        scratch_types=[
            pltpu.VMEM((NB, _B), jnp.int32),
            pltpu.VMEM((NB, _B), jnp.int32),
            pltpu.VMEM((2, _B, QW), jnp.float32),
            pltpu.VMEM((2, _B, KVW), jnp.float32),
            pltpu.VMEM((2, _B, RW), jnp.float32),
            pltpu.VMEM((ZC, RW), jnp.float32),
            pltpu.VMEM((NH, EDIM), jnp.float32),
            pltpu.VMEM_SHARED((n_pad, RW), jnp.float32),
            pltpu.SemaphoreType.DMA,
            pltpu.SemaphoreType.DMA,
            pltpu.SemaphoreType.DMA,
            pltpu.SemaphoreType.DMA,
        ],
    )
    def edge_kernel(qt_hbm, kvt_hbm, row_hbm, col_hbm, wct_hbm, out_hbm,
                    rix_all, cix_all, qrows2, kvrows2, contrib2, stage, wct_v,
                    acc_sh, sg0, sg1, ss0, ss1):
        cid = lax.axis_index("c")
        sid = lax.axis_index("s")
        wid = sid * _NC + cid
        lane_i = lax.iota(jnp.int32, _L)
        lane = lane_i.astype(jnp.float32)
        sg = [sg0, sg1]
        ss = [ss0, ss1]

        pltpu.sync_copy(wct_hbm, wct_v)
        wct = [[wct_v[hh, pl.ds(j * _L, _L)] for j in range(NEC)]
               for hh in range(NH)]
        offs = [(lane + float(j * _L)) * step for j in range(NEC)]

        pltpu.sync_copy(row_hbm.at[wid], rix_all)
        pltpu.sync_copy(col_hbm.at[wid], cix_all)

        def zbody(i, _):
            r = i // (RW // _L)
            c = i % (RW // _L)
            stage[r, pl.ds(c * _L, _L)] = jnp.zeros((_L,), jnp.float32)
            return 0
        lax.fori_loop(0, ZC * (RW // _L), zbody, 0)
        tb = sid * RPT
        for k in range(NZ):
            pltpu.sync_copy(stage, acc_sh.at[pl.ds(tb + k * ZC, ZC)])
        plsc.subcore_barrier()

        def gathers(bi, pp):
            return (pltpu.make_async_copy(qt_hbm.at[rix_all.at[bi]],
                                          qrows2.at[pp], sg[pp]),
                    pltpu.make_async_copy(kvt_hbm.at[cix_all.at[bi]],
                                          kvrows2.at[pp], sg[pp]))

        def scatter(bi, pp):
            return pltpu.make_async_copy(contrib2.at[pp],
                                         acc_sh.at[rix_all.at[bi]], ss[pp])

        def issue_gathers(bi, pp):
            for c in gathers(bi, pp):
                c.start()

        def compute(pp):
            qr = qrows2.at[pp]
            kvr = kvrows2.at[pp]
            ctr = contrib2.at[pp]

            def unpk(w):
                return plsc.unpack(plsc.bitcast(w, jnp.bfloat16),
                                   format=plsc.PackFormat.INTERLEAVED)

            @plsc.parallel_loop(0, _B, 1, unroll=4)
            def edge(i):
                qe = qr[i, pl.ds(NP * _L, _L)]
                ke = kvr[i, pl.ds(NP * _L, _L)]
                dist = jnp.sum(qe * ke, axis=0)
                rbf = []
                for j in range(NEC):
                    d = dist - offs[j]
                    rbf.append(jnp.exp(coeff * (d * d)))
                qk = []
                for pr in range(NP):
                    qa_, qb_ = unpk(qr[i, pl.ds(pr * _L, _L)])
                    ka_, kb_ = unpk(kvr[i, pl.ds(pr * _L, _L)])
                    qk.append(qa_ * ka_)
                    qk.append(qb_ * kb_)
                den = jnp.zeros((_L,), jnp.float32)
                wbs = []
                for hh in range(NH):
                    t = qk[hh] + rbf[0] * wct[hh][0]
                    for j in range(1, NEC):
                        t = t + rbf[j] * wct[hh][j]
                    s = jnp.sum(t, axis=0)
                    wb = jnp.exp(jnp.full((_L,), s))
                    wbs.append(wb)
                    den = jnp.where(lane_i == hh, wb, den)
                for pr in range(NP):
                    va_, vb_ = unpk(kvr[i, pl.ds((NP + 1 + pr) * _L, _L)])
                    ctr[i, pl.ds(2 * pr * _L, _L)] = va_ * wbs[2 * pr]
                    ctr[i, pl.ds((2 * pr + 1) * _L, _L)] = vb_ * wbs[2 * pr + 1]
                ctr[i, pl.ds(NH * _L, _L)] = den

        issue_gathers(0, 0)

        def group(g, _):
            for pp in range(2):
                bi = 2 * g + pp

                @pl.when(bi < NB)
                def _():
                    @pl.when(bi + 1 < NB)
                    def _():
                        issue_gathers(bi + 1, 1 - pp)
                    for c in gathers(bi, pp):
                        c.wait()

                    @pl.when(bi >= 2)
                    def _():
                        scatter(bi - 2, pp).wait()
                    compute(pp)
                    scatter(bi, pp).start(add=True)
            return 0
        lax.fori_loop(0, (NB + 1) // 2, group, 0)
        scatter(NB - 2, (NB - 2) % 2).wait()
        scatter(NB - 1, (NB - 1) % 2).wait()

        plsc.subcore_barrier()
        for k in range(NZ):
            pltpu.sync_copy(acc_sh.at[pl.ds(tb + k * ZC, ZC)], stage)
            pltpu.sync_copy(stage, out_hbm.at[cid, pl.ds(tb + k * ZC, ZC)])

    return edge_kernel(qt, kvt, row, col, WcTp)


def _tc_output(acc_a, acc_b, Wo, bo, W1, b1, W2, b2, Erep, n_rows, NH):
    C = Wo.shape[1]
    HD = Wo.shape[0]
    C2 = W1.shape[1]
    RW = acc_a.shape[2]
    NCQ = acc_a.shape[0]
    NHW = NH * _L

    def body(acca_ref, accb_ref, wo_ref, bo_ref, w1_ref, b1_ref, w2_ref,
             b2_ref, erep_ref, out_ref):
        sa = jnp.sum(acca_ref[...], axis=0)
        sb = jnp.sum(accb_ref[...], axis=0)
        num = jnp.concatenate([sa[:, :NHW], sb[:, :NHW]], axis=1)
        den = jnp.concatenate([sa[:, NHW:NHW + NH], sb[:, NHW:NHW + NH]],
                              axis=1) + 1e-16
        den_wide = jnp.dot(den, erep_ref[...],
                           preferred_element_type=jnp.float32)
        agg = num / den_wide
        x = jnp.dot(agg, wo_ref[...], preferred_element_type=jnp.float32) + bo_ref[...]
        hid = jnp.dot(x, w1_ref[...], preferred_element_type=jnp.float32) + b1_ref[...]
        hid = hid * (1.0 / (1.0 + jnp.exp(-hid)))
        out_ref[...] = jnp.dot(hid, w2_ref[...],
                               preferred_element_type=jnp.float32) + b2_ref[...]

    grid = (n_rows // _ROW_BLK,)
    full = lambda s: pl.BlockSpec(s, lambda i: (0,) * len(s))
    accs = pl.BlockSpec((NCQ, _ROW_BLK, RW), lambda i: (0, i, 0))
    return pl.pallas_call(
        body,
        grid=grid,
        in_specs=[accs, accs,
                  full((HD, C)), full((1, C)), full((C, C2)), full((1, C2)),
                  full((C2, C)), full((1, C)), full((2 * NH, HD))],
        out_specs=pl.BlockSpec((_ROW_BLK, C), lambda i: (i, 0)),
        out_shape=jax.ShapeDtypeStruct((n_rows, C), jnp.float32),
    )(acc_a, acc_b, Wo, bo.reshape(1, -1), W1, b1.reshape(1, -1), W2,
      b2.reshape(1, -1), Erep)


def kernel(pos, h, edge_index, pos_p, p, W_edge, b_edge, Wq, bq, Wk, bk,
           Wv, bv, We, be, Wo, bo, W1, b1, W2, b2):
    N, C = h.shape
    M = p.shape[0]
    E = edge_index.shape[1]
    EDIM = W_edge.shape[0]
    H = We.shape[1]
    DH = Wq.shape[1] // H
    NH = H // 2

    scale = 1.0 / float(np.sqrt(DH))
    Wq_s = Wq * scale
    bq_s = bq * scale
    WcT = (W_edge @ We).T
    offs_np = np.linspace(0.0, 30.0, EDIM)
    step = float(offs_np[1] - offs_np[0])
    coeff = float(-0.5 / step ** 2)

    qa, qb, kva, kvb = _tc_build_tables(h, pos, p, pos_p, Wq_s, bq_s, Wk, bk,
                                        Wv, bv, NH)

    tile = _NW * _B
    E_pad = ((E + tile - 1) // tile) * tile
    row = edge_index[0]
    col = edge_index[1]
    if E_pad != E:
        row = jnp.concatenate([row, jnp.full((E_pad - E,), N, jnp.int32)])
        col = jnp.concatenate([col, jnp.zeros((E_pad - E,), jnp.int32)])
    row = row.reshape(_NW, E_pad // (_NW * _B), _B)
    col = col.reshape(_NW, E_pad // (_NW * _B), _B)
    if E_pad == E and N % _NS == 0:
        n_pad = N
    else:
        n_pad = ((N + 1 + 16 * _NS - 1) // (16 * _NS)) * (16 * _NS)

    acc_a = _sc_edge_pass(qa, kva, row, col, WcT[:NH], step, coeff, n_pad, NH)
    acc_b = _sc_edge_pass(qb, kvb, row, col, WcT[NH:], step, coeff, n_pad, NH)

    lane_h = jnp.arange(H * DH) // DH
    Erep = (lane_h[None, :] == jnp.arange(H)[:, None]).astype(jnp.float32)
    if n_pad != N:
        acc_a = acc_a[:, :N]
        acc_b = acc_b[:, :N]
    return _tc_output(acc_a, acc_b, Wo, bo, W1, b1, W2, b2, Erep, N, NH)
